# baseline (speedup 1.0000x reference)
import functools

import jax
import jax.numpy as jnp
from jax import lax
from jax.experimental import pallas as pl
from jax.experimental.pallas import tpu as pltpu
from jax.experimental.pallas import tpu_sc as plsc

NC, NS, L = 2, 16, 16
NW = NC * NS
K = 128
NB = 3

_SC_PARAMS = pltpu.CompilerParams(use_tc_tiling_on_sc=False)


def _sc_mesh():
    return plsc.VectorSubcoreMesh(
        core_axis_name="c", subcore_axis_name="s", num_cores=NC, num_subcores=NS
    )


def _edge_scatter(hs_pad, src3, dst3, zacc, n_acc, feat, cpt):
    rows_pt = n_acc // NS
    plane = jax.ShapeDtypeStruct((n_acc, feat), jnp.float32)

    scratch = (
        [pltpu.VMEM((cpt, K), jnp.int32)] * 2
        + [pltpu.VMEM((K, feat), jnp.float32)] * NB
        + [pltpu.SemaphoreType.DMA] * (2 * NB + 1)
        + [pltpu.VMEM_SHARED((n_acc, feat), jnp.float32)] * 2
    )

    @functools.partial(
        pl.kernel,
        out_type=[plane, plane],
        mesh=_sc_mesh(),
        scratch_types=scratch,
        compiler_params=_SC_PARAMS,
    )
    def k(hs_hbm, src_hbm, dst_hbm, z_hbm, out0, out1, src_v, dst_v, *rest):
        rb = rest[:NB]
        gsem = rest[NB : 2 * NB]
        ssem = rest[2 * NB : 3 * NB]
        isem = rest[3 * NB]
        acc = rest[3 * NB + 1]
        hsc = rest[3 * NB + 2]
        cid = lax.axis_index("c")
        sid = lax.axis_index("s")
        wid = sid * NC + cid
        r0 = sid * rows_pt
        pltpu.async_copy(src_hbm.at[wid], src_v, isem)
        pltpu.async_copy(dst_hbm.at[wid], dst_v, isem)
        pltpu.sync_copy(hs_hbm.at[pl.ds(r0, rows_pt)], hsc.at[pl.ds(r0, rows_pt)])
        pltpu.sync_copy(z_hbm.at[pl.ds(r0, rows_pt)], acc.at[pl.ds(r0, rows_pt)])
        pltpu.make_async_copy(src_hbm.at[wid], src_v, isem).wait()
        pltpu.make_async_copy(dst_hbm.at[wid], dst_v, isem).wait()
        plsc.subcore_barrier()

        dummy = z_hbm.at[pl.ds(0, K)]
        tabs = [hs_hbm if u == 0 else hsc for u in range(NB)]
        pltpu.async_copy(tabs[0].at[src_v.at[0]], rb[0], gsem[0])

        def ring(q, c):
            for u in range(NB):
                j = q * NB + u
                o = (u + 1) % NB
                pltpu.make_async_copy(dummy, rb[u], gsem[u]).wait()
                pltpu.async_copy(rb[u], acc.at[dst_v.at[j]], ssem[u], add=True)

                @pl.when(j >= NB - 1)
                def _():
                    pltpu.make_async_copy(dummy, rb[o], ssem[o]).wait()

                @pl.when(j + 1 < cpt)
                def _():
                    pltpu.async_copy(tabs[o].at[src_v.at[j + 1]], rb[o], gsem[o])
            return c

        lax.fori_loop(0, cpt // NB, ring, 0)
        for d in range(1, NB):
            pltpu.make_async_copy(dummy, rb[(cpt - d) % NB], ssem[(cpt - d) % NB]).wait()
        plsc.subcore_barrier()

        @pl.when(cid == 0)
        def _():
            pltpu.sync_copy(acc.at[pl.ds(r0, rows_pt)], out0.at[pl.ds(r0, rows_pt)])

        @pl.when(cid == 1)
        def _():
            pltpu.sync_copy(acc.at[pl.ds(r0, rows_pt)], out1.at[pl.ds(r0, rows_pt)])

    return k(hs_pad, src3, dst3, zacc)


DEG_W = 16
DF = 16


def _deg_count(dst3, ones_rows, zdeg, n_acc, cpt):
    rows_pt = n_acc // NS
    plane = jax.ShapeDtypeStruct((n_acc, DF), jnp.float32)

    @functools.partial(
        pl.kernel,
        out_type=[plane, plane],
        mesh=_sc_mesh(),
        scratch_types=[
            pltpu.VMEM((cpt, K), jnp.int32),
            pltpu.VMEM((K, DF), jnp.float32),
            pltpu.SemaphoreType.DMA,
            pltpu.SemaphoreType.DMA,
            pltpu.VMEM_SHARED((n_acc, DF), jnp.float32),
        ],
        compiler_params=_SC_PARAMS,
    )
    def k(dst_hbm, ones_hbm, z_hbm, out0, out1, dst_v, ones_v, isem, ssem, acc):
        cid = lax.axis_index("c")
        sid = lax.axis_index("s")
        wid = sid * NC + cid
        r0 = sid * rows_pt
        pltpu.async_copy(dst_hbm.at[wid], dst_v, isem)
        pltpu.sync_copy(ones_hbm, ones_v)
        pltpu.sync_copy(z_hbm.at[pl.ds(r0, rows_pt)], acc.at[pl.ds(r0, rows_pt)])
        pltpu.make_async_copy(dst_hbm.at[wid], dst_v, isem).wait()
        plsc.subcore_barrier()

        dummy = z_hbm.at[pl.ds(0, K)]

        def fire(j, c):
            pltpu.async_copy(ones_v, acc.at[dst_v.at[j]], ssem, add=True)

            @pl.when(j >= DEG_W)
            def _():
                pltpu.make_async_copy(dummy, ones_v, ssem).wait()

            return c

        lax.fori_loop(0, cpt, fire, 0)

        def drain(j, c):
            pltpu.make_async_copy(dummy, ones_v, ssem).wait()
            return c

        lax.fori_loop(0, min(DEG_W, cpt), drain, 0)
        plsc.subcore_barrier()

        @pl.when(cid == 0)
        def _():
            pltpu.sync_copy(acc.at[pl.ds(r0, rows_pt)], out0.at[pl.ds(r0, rows_pt)])

        @pl.when(cid == 1)
        def _():
            pltpu.sync_copy(acc.at[pl.ds(r0, rows_pt)], out1.at[pl.ds(r0, rows_pt)])

    return k(dst3, ones_rows, zdeg)


def _tc_matmul(x, W, n_acc):
    n, h = x.shape[0], W.shape[1]
    tail = n_acc - n

    def body(xr, wr, outr):
        outr[pl.ds(0, n), :] = jnp.dot(
            xr[...], wr[...], preferred_element_type=jnp.float32
        )
        outr[pl.ds(n, tail), :] = jnp.zeros((tail, h), jnp.float32)

    return pl.pallas_call(
        body, out_shape=jax.ShapeDtypeStruct((n_acc, h), jnp.float32)
    )(x, W)


def _tc_first(d0, d1, mm):
    n_acc, h = mm.shape

    def body(d0r, d1r, mmr, hsr, disr):
        dis = lax.rsqrt(d0r[...][:, 0:1] + d1r[...][:, 0:1] + 1.0)
        hsr[...] = mmr[...] * dis
        disr[...] = dis

    return pl.pallas_call(
        body,
        out_shape=[
            jax.ShapeDtypeStruct((n_acc, h), jnp.float32),
            jax.ShapeDtypeStruct((n_acc, 1), jnp.float32),
        ],
    )(d0, d1, mm)


def _tc_mid(p0, p1, hs_prev, dis, b2d, W):
    n, h = hs_prev.shape[0], W.shape[1]

    def body(p0r, p1r, hpr, disr, br, wr, outr):
        dis = disr[...]
        hh = jnp.maximum(dis * (p0r[...] + p1r[...] + hpr[...]) + br[...], 0.0)
        outr[...] = jnp.dot(hh, wr[...], preferred_element_type=jnp.float32) * dis

    return pl.pallas_call(
        body, out_shape=jax.ShapeDtypeStruct((n, h), jnp.float32)
    )(p0, p1, hs_prev, dis, b2d, W)


def _tc_final(p0, p1, hs3, dis, b3, Wc1, bc1, Wc2, bc2, n_out):
    c = Wc2.shape[1]

    def body(p0r, p1r, hpr, disr, br, w1r, b1r, w2r, b2r, outr):
        hh = jnp.maximum(
            disr[...][0:n_out]
            * (p0r[...][0:n_out] + p1r[...][0:n_out] + hpr[...][0:n_out])
            + br[...],
            0.0,
        )
        m = jnp.maximum(
            jnp.dot(hh, w1r[...], preferred_element_type=jnp.float32) + b1r[...], 0.0
        )
        logits = jnp.dot(m, w2r[...], preferred_element_type=jnp.float32) + b2r[...]
        mx = jnp.max(logits, axis=1, keepdims=True)
        lse = mx + jnp.log(jnp.sum(jnp.exp(logits - mx), axis=1, keepdims=True))
        outr[...] = logits - lse

    return pl.pallas_call(
        body, out_shape=jax.ShapeDtypeStruct((n_out, c), jnp.float32)
    )(p0, p1, hs3, dis, b3, Wc1, bc1, Wc2, bc2)


def kernel(x, edge_index, W1, b1, W2, b2, W3, b3, Wc1, bc1, Wc2, bc2):
    N, _ = x.shape
    H = W1.shape[1]
    E = edge_index.shape[1]
    n_acc = -(-(N + 1) // (NS * 8)) * (NS * 8)

    src = edge_index[0]
    dst = edge_index[1]
    cpt = -(-E // (NW * K))
    cpt = -(-cpt // NB) * NB
    e_pad = cpt * NW * K
    pad = e_pad - E
    src_p = jnp.concatenate([src, jnp.zeros((pad,), jnp.int32)]).reshape(NW, cpt, K)
    dst_p = jnp.concatenate([dst, jnp.full((pad,), N, jnp.int32)]).reshape(NW, cpt, K)

    z_feat = jnp.zeros((n_acc, H), jnp.float32)
    z_deg = jnp.zeros((n_acc, DF), jnp.float32)
    ones_rows = jnp.ones((K, DF), jnp.float32)

    mm1 = _tc_matmul(x, W1, n_acc)
    d0, d1 = _deg_count(dst_p, ones_rows, z_deg, n_acc, cpt)
    hs1, dis = _tc_first(d0, d1, mm1)

    P0, P1 = _edge_scatter(hs1, src_p, dst_p, z_feat, n_acc, H, cpt)
    hs2 = _tc_mid(P0, P1, hs1, dis, b1.reshape(1, H), W2)

    P0, P1 = _edge_scatter(hs2, src_p, dst_p, z_feat, n_acc, H, cpt)
    hs3 = _tc_mid(P0, P1, hs2, dis, b2.reshape(1, H), W3)

    P0, P1 = _edge_scatter(hs3, src_p, dst_p, z_feat, n_acc, H, cpt)
    return _tc_final(
        P0, P1, hs3, dis, b3.reshape(1, H),
        Wc1, bc1.reshape(1, -1), Wc2, bc2.reshape(1, -1), N,
    )


# speedup vs baseline: 1.6839x; 1.6839x over previous
import functools

import jax
import jax.numpy as jnp
from jax import lax
from jax.experimental import pallas as pl
from jax.experimental.pallas import tpu as pltpu
from jax.experimental.pallas import tpu_sc as plsc

NC, NS, L = 2, 16, 16
NW = NC * NS
K = 128
NB = 3

_SC_PARAMS = pltpu.CompilerParams(use_tc_tiling_on_sc=False)


def _sc_mesh():
    return plsc.VectorSubcoreMesh(
        core_axis_name="c", subcore_axis_name="s", num_cores=NC, num_subcores=NS
    )


def _edge_scatter(hs_pad, src3, dst3, zacc, n_acc, feat, cpt):
    rows_pt = n_acc // NS
    plane = jax.ShapeDtypeStruct((n_acc, feat), jnp.float32)

    scratch = (
        [pltpu.VMEM((cpt, K), jnp.int32)] * 2
        + [pltpu.VMEM((K, feat), jnp.float32)] * NB
        + [pltpu.SemaphoreType.DMA] * (2 * NB + 1)
        + [pltpu.VMEM_SHARED((n_acc, feat), jnp.float32)] * 2
    )

    @functools.partial(
        pl.kernel,
        out_type=[plane, plane],
        mesh=_sc_mesh(),
        scratch_types=scratch,
        compiler_params=_SC_PARAMS,
    )
    def k(hs_hbm, src_hbm, dst_hbm, z_hbm, out0, out1, src_v, dst_v, *rest):
        rb = rest[:NB]
        gsem = rest[NB : 2 * NB]
        ssem = rest[2 * NB : 3 * NB]
        isem = rest[3 * NB]
        acc = rest[3 * NB + 1]
        hsc = rest[3 * NB + 2]
        cid = lax.axis_index("c")
        sid = lax.axis_index("s")
        wid = sid * NC + cid
        r0 = sid * rows_pt
        pltpu.async_copy(src_hbm.at[wid], src_v, isem)
        pltpu.async_copy(dst_hbm.at[wid], dst_v, isem)
        pltpu.sync_copy(hs_hbm.at[pl.ds(r0, rows_pt)], hsc.at[pl.ds(r0, rows_pt)])
        pltpu.sync_copy(z_hbm.at[pl.ds(r0, rows_pt)], acc.at[pl.ds(r0, rows_pt)])
        pltpu.make_async_copy(src_hbm.at[wid], src_v, isem).wait()
        pltpu.make_async_copy(dst_hbm.at[wid], dst_v, isem).wait()
        plsc.subcore_barrier()

        dummy = z_hbm.at[pl.ds(0, K)]
        tabs = [hsc] * NB
        pltpu.async_copy(tabs[0].at[src_v.at[0]], rb[0], gsem[0])

        def ring(q, c):
            for u in range(NB):
                j = q * NB + u
                o = (u + 1) % NB
                pltpu.make_async_copy(dummy, rb[u], gsem[u]).wait()
                pltpu.async_copy(rb[u], acc.at[dst_v.at[j]], ssem[u], add=True)

                @pl.when(j >= NB - 1)
                def _():
                    pltpu.make_async_copy(dummy, rb[o], ssem[o]).wait()

                @pl.when(j + 1 < cpt)
                def _():
                    pltpu.async_copy(tabs[o].at[src_v.at[j + 1]], rb[o], gsem[o])
            return c

        lax.fori_loop(0, cpt // NB, ring, 0)
        for d in range(1, NB):
            pltpu.make_async_copy(dummy, rb[(cpt - d) % NB], ssem[(cpt - d) % NB]).wait()
        plsc.subcore_barrier()

        @pl.when(cid == 0)
        def _():
            pltpu.sync_copy(acc.at[pl.ds(r0, rows_pt)], out0.at[pl.ds(r0, rows_pt)])

        @pl.when(cid == 1)
        def _():
            pltpu.sync_copy(acc.at[pl.ds(r0, rows_pt)], out1.at[pl.ds(r0, rows_pt)])

    return k(hs_pad, src3, dst3, zacc)


DEG_W = 16
DF = 16


def _deg_count(dst3, ones_rows, zdeg, n_acc, cpt):
    rows_pt = n_acc // NS
    plane = jax.ShapeDtypeStruct((n_acc, DF), jnp.float32)

    @functools.partial(
        pl.kernel,
        out_type=[plane, plane],
        mesh=_sc_mesh(),
        scratch_types=[
            pltpu.VMEM((cpt, K), jnp.int32),
            pltpu.VMEM((K, DF), jnp.float32),
            pltpu.SemaphoreType.DMA,
            pltpu.SemaphoreType.DMA,
            pltpu.VMEM_SHARED((n_acc, DF), jnp.float32),
        ],
        compiler_params=_SC_PARAMS,
    )
    def k(dst_hbm, ones_hbm, z_hbm, out0, out1, dst_v, ones_v, isem, ssem, acc):
        cid = lax.axis_index("c")
        sid = lax.axis_index("s")
        wid = sid * NC + cid
        r0 = sid * rows_pt
        pltpu.async_copy(dst_hbm.at[wid], dst_v, isem)
        pltpu.sync_copy(ones_hbm, ones_v)
        pltpu.sync_copy(z_hbm.at[pl.ds(r0, rows_pt)], acc.at[pl.ds(r0, rows_pt)])
        pltpu.make_async_copy(dst_hbm.at[wid], dst_v, isem).wait()
        plsc.subcore_barrier()

        dummy = z_hbm.at[pl.ds(0, K)]

        def fire(j, c):
            pltpu.async_copy(ones_v, acc.at[dst_v.at[j]], ssem, add=True)

            @pl.when(j >= DEG_W)
            def _():
                pltpu.make_async_copy(dummy, ones_v, ssem).wait()

            return c

        lax.fori_loop(0, cpt, fire, 0)

        def drain(j, c):
            pltpu.make_async_copy(dummy, ones_v, ssem).wait()
            return c

        lax.fori_loop(0, min(DEG_W, cpt), drain, 0)
        plsc.subcore_barrier()

        @pl.when(cid == 0)
        def _():
            pltpu.sync_copy(acc.at[pl.ds(r0, rows_pt)], out0.at[pl.ds(r0, rows_pt)])

        @pl.when(cid == 1)
        def _():
            pltpu.sync_copy(acc.at[pl.ds(r0, rows_pt)], out1.at[pl.ds(r0, rows_pt)])

    return k(dst3, ones_rows, zdeg)


def _tc_matmul(x, W, n_acc):
    n, h = x.shape[0], W.shape[1]
    tail = n_acc - n

    def body(xr, wr, outr):
        outr[pl.ds(0, n), :] = jnp.dot(
            xr[...], wr[...], preferred_element_type=jnp.float32
        )
        outr[pl.ds(n, tail), :] = jnp.zeros((tail, h), jnp.float32)

    return pl.pallas_call(
        body, out_shape=jax.ShapeDtypeStruct((n_acc, h), jnp.float32)
    )(x, W)


def _tc_first(d0, d1, mm):
    n_acc, h = mm.shape

    def body(d0r, d1r, mmr, hsr, disr):
        dis = lax.rsqrt(d0r[...][:, 0:1] + d1r[...][:, 0:1] + 1.0)
        hsr[...] = mmr[...] * dis
        disr[...] = dis

    return pl.pallas_call(
        body,
        out_shape=[
            jax.ShapeDtypeStruct((n_acc, h), jnp.float32),
            jax.ShapeDtypeStruct((n_acc, 1), jnp.float32),
        ],
    )(d0, d1, mm)


def _tc_mid(p0, p1, hs_prev, dis, b2d, W):
    n, h = hs_prev.shape[0], W.shape[1]

    def body(p0r, p1r, hpr, disr, br, wr, outr):
        dis = disr[...]
        hh = jnp.maximum(dis * (p0r[...] + p1r[...] + hpr[...]) + br[...], 0.0)
        outr[...] = jnp.dot(hh, wr[...], preferred_element_type=jnp.float32) * dis

    return pl.pallas_call(
        body, out_shape=jax.ShapeDtypeStruct((n, h), jnp.float32)
    )(p0, p1, hs_prev, dis, b2d, W)


def _tc_final(p0, p1, hs3, dis, b3, Wc1, bc1, Wc2, bc2, n_out):
    c = Wc2.shape[1]

    def body(p0r, p1r, hpr, disr, br, w1r, b1r, w2r, b2r, outr):
        hh = jnp.maximum(
            disr[...][0:n_out]
            * (p0r[...][0:n_out] + p1r[...][0:n_out] + hpr[...][0:n_out])
            + br[...],
            0.0,
        )
        m = jnp.maximum(
            jnp.dot(hh, w1r[...], preferred_element_type=jnp.float32) + b1r[...], 0.0
        )
        logits = jnp.dot(m, w2r[...], preferred_element_type=jnp.float32) + b2r[...]
        mx = jnp.max(logits, axis=1, keepdims=True)
        lse = mx + jnp.log(jnp.sum(jnp.exp(logits - mx), axis=1, keepdims=True))
        outr[...] = logits - lse

    return pl.pallas_call(
        body, out_shape=jax.ShapeDtypeStruct((n_out, c), jnp.float32)
    )(p0, p1, hs3, dis, b3, Wc1, bc1, Wc2, bc2)


def kernel(x, edge_index, W1, b1, W2, b2, W3, b3, Wc1, bc1, Wc2, bc2):
    N, _ = x.shape
    H = W1.shape[1]
    E = edge_index.shape[1]
    n_acc = -(-(N + 1) // (NS * 8)) * (NS * 8)

    src = edge_index[0]
    dst = edge_index[1]
    cpt = -(-E // (NW * K))
    cpt = -(-cpt // NB) * NB
    e_pad = cpt * NW * K
    pad = e_pad - E
    src_p = jnp.concatenate([src, jnp.zeros((pad,), jnp.int32)]).reshape(NW, cpt, K)
    dst_p = jnp.concatenate([dst, jnp.full((pad,), N, jnp.int32)]).reshape(NW, cpt, K)

    z_feat = jnp.zeros((n_acc, H), jnp.float32)
    z_deg = jnp.zeros((n_acc, DF), jnp.float32)
    ones_rows = jnp.ones((K, DF), jnp.float32)

    mm1 = _tc_matmul(x, W1, n_acc)
    d0, d1 = _deg_count(dst_p, ones_rows, z_deg, n_acc, cpt)
    hs1, dis = _tc_first(d0, d1, mm1)

    P0, P1 = _edge_scatter(hs1, src_p, dst_p, z_feat, n_acc, H, cpt)
    hs2 = _tc_mid(P0, P1, hs1, dis, b1.reshape(1, H), W2)

    P0, P1 = _edge_scatter(hs2, src_p, dst_p, z_feat, n_acc, H, cpt)
    hs3 = _tc_mid(P0, P1, hs2, dis, b2.reshape(1, H), W3)

    P0, P1 = _edge_scatter(hs3, src_p, dst_p, z_feat, n_acc, H, cpt)
    return _tc_final(
        P0, P1, hs3, dis, b3.reshape(1, H),
        Wc1, bc1.reshape(1, -1), Wc2, bc2.reshape(1, -1), N,
    )
